# Initial kernel scaffold; baseline (speedup 1.0000x reference)
import jax
import jax.numpy as jnp
from jax.experimental import pallas as pl


def kernel(x, x_v, in_proj_weight, out_proj_weight):
    raise NotImplementedError("write your pallas kernel here")


# speedup vs baseline: 7.2108x; 7.2108x over previous
import functools

import jax
import jax.numpy as jnp
import numpy as np
from jax import lax
from jax.experimental import pallas as pl
from jax.experimental.pallas import tpu as pltpu
from jax.experimental.pallas import tpu_sc as plsc

_B, _N, _E, _H, _K = 8, 2048, 256, 8, 8
_HD = _E // _H

_F32 = jnp.float32
_HIGH = lax.Precision.HIGHEST

_TNB = 512


def _knn_body(xvr_ref, xvc_ref, out_ref):
    b = pl.program_id(0)
    xr = xvr_ref[0]
    xc = xvc_ref[0]
    sqr = jnp.sum(xr * xr, axis=1, keepdims=True)
    ones8 = jnp.ones((8, 128), _F32)
    sqc = lax.dot_general(ones8, xc * xc, (((1,), (1,)), ((), ())),
                          preferred_element_type=_F32, precision=_HIGH)
    sqc_row = sqc[0:1, :]
    g = lax.dot_general(xr, xc, (((1,), (1,)), ((), ())),
                        preferred_element_type=_F32,
                        precision=lax.Precision.DEFAULT)
    d = (sqr + sqc_row) - 2.0 * g
    iota = lax.broadcasted_iota(jnp.int32, (_TNB, _N), 1)
    for t in range(_K):
        m = jnp.min(d, axis=1, keepdims=True)
        cand = jnp.where(d == m, iota, jnp.int32(1 << 30))
        sel = jnp.min(cand, axis=1)
        out_ref[0, t, :] = sel + b * _N
        d = jnp.where(iota == sel[:, None], jnp.float32(jnp.inf), d)


def _knn_idx_global(xvp):
    return pl.pallas_call(
        _knn_body,
        grid=(_B, _N // _TNB),
        in_specs=[
            pl.BlockSpec((1, _TNB, 128), lambda b, i: (b, i, 0)),
            pl.BlockSpec((1, _N, 128), lambda b, i: (b, 0, 0)),
        ],
        out_specs=pl.BlockSpec((1, _K, _TNB), lambda b, i: (b, 0, i)),
        out_shape=jax.ShapeDtypeStruct((_B, _K, _N), jnp.int32),
    )(xvp, xvp)


_TNP = 512


def _proj_body(x_ref, w_ref, out_ref):
    out_ref[:] = jnp.dot(x_ref[:], w_ref[:],
                         preferred_element_type=_F32, precision=_HIGH)


def _proj_kv(x2, wkvT):
    return pl.pallas_call(
        _proj_body,
        grid=(_B * _N // _TNP,),
        in_specs=[
            pl.BlockSpec((_TNP, _E), lambda i: (i, 0)),
            pl.BlockSpec((_E, 2 * _E), lambda i: (0, 0)),
        ],
        out_specs=pl.BlockSpec((_TNP, 2 * _E), lambda i: (i, 0)),
        out_shape=jax.ShapeDtypeStruct((_B * _N, 2 * _E), _F32),
    )(x2, wkvT)


_SC_CHUNK = 128
_ROWS = _B * _N * _K


def _gather_rows_sc(xkv, gidx):
    info = plsc.get_sparse_core_info()
    nw = info.num_cores * info.num_subcores
    rpw = _ROWS // nw
    mesh = plsc.VectorSubcoreMesh(core_axis_name="c", subcore_axis_name="s")

    @functools.partial(
        pl.kernel,
        mesh=mesh,
        out_type=jax.ShapeDtypeStruct((_ROWS, 2 * _E), _F32),
        scratch_types=[
            pltpu.VMEM((_SC_CHUNK,), jnp.int32),
            pltpu.VMEM((_SC_CHUNK, 2 * _E), _F32),
            pltpu.SemaphoreType.DMA,
        ],
    )
    def k(xkv_hbm, gidx_hbm, out_hbm, idx_v, rows_v, sem):
        wid = lax.axis_index("s") * info.num_cores + lax.axis_index("c")
        base0 = wid * rpw

        def body(i, carry):
            base = base0 + i * _SC_CHUNK
            pltpu.sync_copy(gidx_hbm.at[pl.ds(base, _SC_CHUNK)], idx_v)
            pltpu.async_copy(xkv_hbm.at[idx_v], rows_v, sem).wait()
            pltpu.sync_copy(rows_v, out_hbm.at[pl.ds(base, _SC_CHUNK)])
            return carry

        lax.fori_loop(0, rpw // _SC_CHUNK, body, 0)

    return k(xkv, gidx)


_TNA = 256


def _attn_body(x_ref, xkv_ref, kvp_ref, wqT_ref, woT_ref, out_ref):
    xr = x_ref[:]
    q = jnp.dot(xr, wqT_ref[:], preferred_element_type=_F32, precision=_HIGH)
    kvp = kvp_ref[:]
    kp3 = kvp[:, :, :_E]
    vp3 = kvp[:, :, _E:]
    lane_h = lax.broadcasted_iota(jnp.int32, (_E, _H), 0) // _HD
    head_h = lax.broadcasted_iota(jnp.int32, (_E, _H), 1)
    hm = (lane_h == head_h).astype(_F32)
    lane_v = lax.broadcasted_iota(jnp.int32, (_H, _E), 1) // _HD
    head_v = lax.broadcasted_iota(jnp.int32, (_H, _E), 0)
    hmT = (lane_v == head_v).astype(_F32)

    s = kp3 * q[:, None, :]
    scores = jnp.dot(s.reshape(_TNA * _K, _E), hm,
                     preferred_element_type=_F32, precision=_HIGH)
    scores = scores / np.sqrt(_HD).astype(np.float32)
    sc3 = scores.reshape(_TNA, _K, _H)
    mx = jnp.max(sc3, axis=1, keepdims=True)
    e = jnp.exp(sc3 - mx)
    a = e / jnp.sum(e, axis=1, keepdims=True)
    attn_exp = jnp.dot(a.reshape(_TNA * _K, _H), hmT,
                       preferred_element_type=_F32, precision=_HIGH)
    o = jnp.sum(attn_exp.reshape(_TNA, _K, _E) * vp3, axis=1)
    o = o - xkv_ref[:, _E:]
    out_ref[:] = xr + jnp.dot(o, woT_ref[:],
                              preferred_element_type=_F32, precision=_HIGH)


def _attn_out(x2, xkv, kvp3, wqT, woutT):
    return pl.pallas_call(
        _attn_body,
        grid=(_B * _N // _TNA,),
        in_specs=[
            pl.BlockSpec((_TNA, _E), lambda i: (i, 0)),
            pl.BlockSpec((_TNA, 2 * _E), lambda i: (i, 0)),
            pl.BlockSpec((_TNA, _K, 2 * _E), lambda i: (i, 0, 0)),
            pl.BlockSpec((_E, _E), lambda i: (0, 0)),
            pl.BlockSpec((_E, _E), lambda i: (0, 0)),
        ],
        out_specs=pl.BlockSpec((_TNA, _E), lambda i: (i, 0)),
        out_shape=jax.ShapeDtypeStruct((_B * _N, _E), _F32),
    )(x2, xkv, kvp3, wqT, woutT)


def kernel(x, x_v, in_proj_weight, out_proj_weight):
    x2 = x.reshape(_B * _N, _E)
    xvp = jnp.pad(x_v, ((0, 0), (0, 0), (0, 128 - 3)))
    Wq, Wk, Wv = jnp.split(in_proj_weight, 3, axis=0)
    wqT = Wq.T
    wkvT = jnp.concatenate([Wk.T, Wv.T], axis=1)
    woutT = out_proj_weight.T

    gidxT = _knn_idx_global(xvp)
    gidx = jnp.transpose(gidxT, (0, 2, 1)).reshape(_ROWS)

    xkv = _proj_kv(x2, wkvT)
    kvp = _gather_rows_sc(xkv, gidx)
    out2 = _attn_out(x2, xkv, kvp.reshape(_B * _N, _K, 2 * _E), wqT, woutT)
    return out2.reshape(_B, _N, _E)


# speedup vs baseline: 12.0954x; 1.6774x over previous
import functools

import jax
import jax.numpy as jnp
import numpy as np
from jax import lax
from jax.experimental import pallas as pl
from jax.experimental.pallas import tpu as pltpu
from jax.experimental.pallas import tpu_sc as plsc

_B, _N, _E, _H, _K = 8, 2048, 256, 8, 8
_HD = _E // _H

_F32 = jnp.float32
_HIGH = lax.Precision.HIGHEST
_DEF = lax.Precision.DEFAULT

_TNB = 512


def _knn_body(xvr_ref, xvc_ref, out_ref):
    b = pl.program_id(0)
    xr = xvr_ref[0]
    xc = xvc_ref[0]
    sqr = jnp.sum(xr * xr, axis=1, keepdims=True)
    ones8 = jnp.ones((8, 128), _F32)
    sqc = lax.dot_general(ones8, xc * xc, (((1,), (1,)), ((), ())),
                          preferred_element_type=_F32, precision=_HIGH)
    sqc_row = sqc[0:1, :]
    g = lax.dot_general(xr, xc, (((1,), (1,)), ((), ())),
                        preferred_element_type=_F32,
                        precision=lax.Precision.DEFAULT)
    d = (sqr + sqc_row) - 2.0 * g
    iota = lax.broadcasted_iota(jnp.int32, (_TNB, _N), 1)
    u = lax.bitcast_convert_type(jnp.maximum(d, 0.0), jnp.int32)
    key = (u & jnp.int32(~2047)) | iota
    for t in range(_K):
        m = jnp.min(key, axis=1, keepdims=True)
        out_ref[0, t, :] = (m[:, 0] & 2047) + b * _N
        key = jnp.where(key == m, jnp.int32(0x7FFFFFFF), key)


def _knn_idx_global(xvp):
    return pl.pallas_call(
        _knn_body,
        grid=(_B, _N // _TNB),
        in_specs=[
            pl.BlockSpec((1, _TNB, 128), lambda b, i: (b, i, 0)),
            pl.BlockSpec((1, _N, 128), lambda b, i: (b, 0, 0)),
        ],
        out_specs=pl.BlockSpec((1, _K, _TNB), lambda b, i: (b, 0, i)),
        out_shape=jax.ShapeDtypeStruct((_B, _K, _N), jnp.int32),
    )(xvp, xvp)


_TNP = 512


def _proj_body(x_ref, w_ref, out_ref):
    out_ref[:] = jnp.dot(x_ref[:], w_ref[:],
                         preferred_element_type=_F32, precision=_DEF)


def _proj_kv(x2, wkvT):
    return pl.pallas_call(
        _proj_body,
        grid=(_B * _N // _TNP,),
        in_specs=[
            pl.BlockSpec((_TNP, _E), lambda i: (i, 0)),
            pl.BlockSpec((_E, 2 * _E), lambda i: (0, 0)),
        ],
        out_specs=pl.BlockSpec((_TNP, 2 * _E), lambda i: (i, 0)),
        out_shape=jax.ShapeDtypeStruct((_B * _N, 2 * _E), _F32),
    )(x2, wkvT)


_SC_CHUNK = 128
_ROWS = _B * _N * _K


def _gather_rows_sc(xkv, gidx):
    info = plsc.get_sparse_core_info()
    nw = info.num_cores * info.num_subcores
    rpw = _ROWS // nw
    mesh = plsc.VectorSubcoreMesh(core_axis_name="c", subcore_axis_name="s")

    @functools.partial(
        pl.kernel,
        mesh=mesh,
        out_type=jax.ShapeDtypeStruct((_ROWS, 2 * _E), _F32),
        scratch_types=[
            pltpu.VMEM((_SC_CHUNK,), jnp.int32),
            pltpu.VMEM((_SC_CHUNK, 2 * _E), _F32),
            pltpu.SemaphoreType.DMA,
        ],
    )
    def k(xkv_hbm, gidx_hbm, out_hbm, idx_v, rows_v, sem):
        wid = lax.axis_index("s") * info.num_cores + lax.axis_index("c")
        base0 = wid * rpw

        def body(i, carry):
            base = base0 + i * _SC_CHUNK
            pltpu.sync_copy(gidx_hbm.at[pl.ds(base, _SC_CHUNK)], idx_v)
            pltpu.async_copy(xkv_hbm.at[idx_v], rows_v, sem).wait()
            pltpu.sync_copy(rows_v, out_hbm.at[pl.ds(base, _SC_CHUNK)])
            return carry

        lax.fori_loop(0, rpw // _SC_CHUNK, body, 0)

    return k(xkv, gidx)


_TNA = 256


def _attn_body(x_ref, xkv_ref, kvp_ref, wqT_ref, woT_ref, out_ref):
    xr = x_ref[:]
    q = jnp.dot(xr, wqT_ref[:], preferred_element_type=_F32, precision=_DEF)
    kvp = kvp_ref[:]
    kp3 = kvp[:, :, :_E]
    vp3 = kvp[:, :, _E:]
    lane_h = lax.broadcasted_iota(jnp.int32, (_E, _H), 0) // _HD
    head_h = lax.broadcasted_iota(jnp.int32, (_E, _H), 1)
    hm = (lane_h == head_h).astype(_F32)
    lane_v = lax.broadcasted_iota(jnp.int32, (_H, _E), 1) // _HD
    head_v = lax.broadcasted_iota(jnp.int32, (_H, _E), 0)
    hmT = (lane_v == head_v).astype(_F32)

    s = kp3 * q[:, None, :]
    scores = jnp.dot(s.reshape(_TNA * _K, _E), hm,
                     preferred_element_type=_F32, precision=_DEF)
    scores = scores / np.sqrt(_HD).astype(np.float32)
    sc3 = scores.reshape(_TNA, _K, _H)
    mx = jnp.max(sc3, axis=1, keepdims=True)
    e = jnp.exp(sc3 - mx)
    a = e / jnp.sum(e, axis=1, keepdims=True)
    attn_exp = jnp.dot(a.reshape(_TNA * _K, _H), hmT,
                       preferred_element_type=_F32, precision=_DEF)
    o = jnp.sum(attn_exp.reshape(_TNA, _K, _E) * vp3, axis=1)
    o = o - xkv_ref[:, _E:]
    out_ref[:] = xr + jnp.dot(o, woT_ref[:],
                              preferred_element_type=_F32, precision=_DEF)


def _attn_out(x2, xkv, kvp3, wqT, woutT):
    return pl.pallas_call(
        _attn_body,
        grid=(_B * _N // _TNA,),
        in_specs=[
            pl.BlockSpec((_TNA, _E), lambda i: (i, 0)),
            pl.BlockSpec((_TNA, 2 * _E), lambda i: (i, 0)),
            pl.BlockSpec((_TNA, _K, 2 * _E), lambda i: (i, 0, 0)),
            pl.BlockSpec((_E, _E), lambda i: (0, 0)),
            pl.BlockSpec((_E, _E), lambda i: (0, 0)),
        ],
        out_specs=pl.BlockSpec((_TNA, _E), lambda i: (i, 0)),
        out_shape=jax.ShapeDtypeStruct((_B * _N, _E), _F32),
    )(x2, xkv, kvp3, wqT, woutT)


def kernel(x, x_v, in_proj_weight, out_proj_weight):
    x2 = x.reshape(_B * _N, _E)
    xvp = jnp.pad(x_v, ((0, 0), (0, 0), (0, 128 - 3)))
    Wq, Wk, Wv = jnp.split(in_proj_weight, 3, axis=0)
    wqT = Wq.T
    wkvT = jnp.concatenate([Wk.T, Wv.T], axis=1)
    woutT = out_proj_weight.T

    gidxT = _knn_idx_global(xvp)
    gidx = jnp.transpose(gidxT, (0, 2, 1)).reshape(_ROWS)

    xkv = _proj_kv(x2, wkvT)
    kvp = _gather_rows_sc(xkv, gidx)
    out2 = _attn_out(x2, xkv, kvp.reshape(_B * _N, _K, 2 * _E), wqT, woutT)
    return out2.reshape(_B, _N, _E)
